# baseline (speedup 1.0000x reference)
import functools

import jax
import jax.numpy as jnp
from jax import lax
from jax.experimental import pallas as pl
from jax.experimental.pallas import tpu as pltpu

_VMEM_LIMIT = 48 * 1024 * 1024
_BF = jnp.bfloat16


def _pw_kernel(x_ref, w_ref, b_ref, o_ref):
    y = jnp.dot(x_ref[...], w_ref[...], preferred_element_type=jnp.float32)
    o_ref[...] = (y + b_ref[...].astype(jnp.float32)).astype(o_ref.dtype)


def _conv1x1(x, w, b):
    N, H, W, Cin = x.shape
    F = w.shape[1]
    M = N * H * W
    xf = x.reshape(M, Cin)
    TM = M // 2 if M % 2 == 0 else M
    grid = (M // TM,)
    out = pl.pallas_call(
        _pw_kernel,
        out_shape=jax.ShapeDtypeStruct((M, F), jnp.float32),
        grid=grid,
        in_specs=[
            pl.BlockSpec((TM, Cin), lambda m: (m, 0)),
            pl.BlockSpec((Cin, F), lambda m: (0, 0)),
            pl.BlockSpec((1, F), lambda m: (0, 0)),
        ],
        out_specs=pl.BlockSpec((TM, F), lambda m: (m, 0)),
        compiler_params=pltpu.CompilerParams(
            dimension_semantics=("parallel",),
            vmem_limit_bytes=_VMEM_LIMIT),
    )(xf, w, b.reshape(1, F))
    return out.reshape(N, H, W, F)


def _pw_upadd_kernel(x_ref, w_ref, b_ref, r_ref, o_ref):
    _, TH2, _, W, Cin = x_ref.shape
    F = w_ref.shape[1]
    W2 = r_ref.shape[2]

    xa = x_ref[0].reshape(TH2 * 2, W, Cin)
    ya = lax.dot_general(xa, w_ref[...],
                         dimension_numbers=(((2,), (0,)), ((), ())),
                         preferred_element_type=jnp.float32)

    r = r_ref[0]
    wf = lax.broadcasted_iota(jnp.int32, (W, W2), 0)
    wc = lax.broadcasted_iota(jnp.int32, (W, W2), 1)
    up = (wc == wf // 2).astype(_BF)
    upb = jnp.broadcast_to(up[None], (TH2, W, W2))
    r_up = lax.dot_general(upb, r,
                           dimension_numbers=(((2,), (1,)), ((0,), (0,))),
                           preferred_element_type=jnp.float32)

    bias = b_ref[...].astype(jnp.float32).reshape(1, 1, 1, F)
    out = ya.reshape(TH2, 2, W, F) + bias + r_up[:, None, :, :]
    o_ref[0] = out.astype(o_ref.dtype)


def _conv1x1_upsample_add(x, w, b, r):
    N, H, W, Cin = x.shape
    F = w.shape[1]
    H2, W2 = H // 2, W // 2

    x5 = x.reshape(N, H2, 2, W, Cin)
    TH2 = H2 // 2 if H2 % 2 == 0 else H2
    grid = (N, H2 // TH2)

    out5 = pl.pallas_call(
        _pw_upadd_kernel,
        out_shape=jax.ShapeDtypeStruct((N, H2, 2, W, F), jnp.float32),
        grid=grid,
        in_specs=[
            pl.BlockSpec((1, TH2, 2, W, Cin), lambda n, i: (n, i, 0, 0, 0)),
            pl.BlockSpec((Cin, F), lambda n, i: (0, 0)),
            pl.BlockSpec((1, F), lambda n, i: (0, 0)),
            pl.BlockSpec((1, TH2, W2, F), lambda n, i: (n, i, 0, 0)),
        ],
        out_specs=pl.BlockSpec((1, TH2, 2, W, F), lambda n, i: (n, i, 0, 0, 0)),
        compiler_params=pltpu.CompilerParams(
            dimension_semantics=("parallel", "parallel"),
            vmem_limit_bytes=_VMEM_LIMIT),
    )(x5, w, b.reshape(1, F), r.reshape(N, H2, W2, F))
    return out5.reshape(N, H, W, F)


def _c3s1_kernel(x_ref, w_ref, b_ref, o_ref, acc_ref, *, TH, W, Cout):
    i = pl.program_id(1)
    bias = b_ref[...].astype(jnp.float32).reshape(1, 1, Cout)
    acc_ref[...] = jnp.broadcast_to(bias, (TH, W, Cout))

    def tap(lhs, k):
        return lax.dot_general(lhs, w_ref[k],
                               dimension_numbers=(((2,), (0,)), ((), ())),
                               preferred_element_type=jnp.float32)

    for dy in range(3):
        rows = x_ref[0, pl.ds(i * TH + dy, TH)]
        acc_ref[...] += tap(rows, 3 * dy + 1)
        acc_ref[:, 1:W] += tap(rows[:, 0:W - 1], 3 * dy + 0)
        acc_ref[:, 0:W - 1] += tap(rows[:, 1:W], 3 * dy + 2)
    o_ref[0] = acc_ref[...].astype(o_ref.dtype)


def _conv3x3_s1(x, w9, b, row_tile):
    N, H, W, Cin = x.shape
    Cout = w9.shape[-1]
    TH = min(row_tile, H)
    xp = jnp.pad(x, ((0, 0), (1, 1), (0, 0), (0, 0)))
    body = functools.partial(_c3s1_kernel, TH=TH, W=W, Cout=Cout)
    return pl.pallas_call(
        body,
        out_shape=jax.ShapeDtypeStruct((N, H, W, Cout), jnp.float32),
        grid=(N, H // TH),
        in_specs=[
            pl.BlockSpec((1, H + 2, W, Cin), lambda n, i: (n, 0, 0, 0)),
            pl.BlockSpec((9, Cin, Cout), lambda n, i: (0, 0, 0)),
            pl.BlockSpec((1, Cout), lambda n, i: (0, 0)),
        ],
        out_specs=pl.BlockSpec((1, TH, W, Cout), lambda n, i: (n, i, 0, 0)),
        scratch_shapes=[pltpu.VMEM((TH, W, Cout), jnp.float32)],
        compiler_params=pltpu.CompilerParams(
            dimension_semantics=("parallel", "arbitrary"),
            vmem_limit_bytes=_VMEM_LIMIT),
    )(xp, w9, b.reshape(1, Cout))


def _c3s2_kernel(x_ref, w_ref, b_ref, o_ref, acc_ref, *,
                 H2, W_in, W_out, Cout, apply_relu):
    x = x_ref[0]
    if apply_relu:
        x = jnp.maximum(x, jnp.zeros_like(x))
    Cin = x.shape[-1]
    x5 = x.reshape(H2, 2, W_in, Cin)

    bias = b_ref[...].astype(jnp.float32).reshape(1, 1, Cout)
    acc_ref[...] = jnp.broadcast_to(bias, (H2, W_in, Cout))

    def tap(lhs, k):
        return lax.dot_general(lhs, w_ref[k],
                               dimension_numbers=(((2,), (0,)), ((), ())),
                               preferred_element_type=jnp.float32)

    def cols(rows, ky, ro, nr):
        acc_ref[ro:ro + nr] += tap(rows, 3 * ky + 1)
        acc_ref[ro:ro + nr, 1:W_in] += tap(rows[:, 0:W_in - 1], 3 * ky + 0)
        acc_ref[ro:ro + nr, 0:W_in - 1] += tap(rows[:, 1:W_in], 3 * ky + 2)

    cols(x5[:, 0], 1, 0, H2)
    cols(x5[:, 1], 2, 0, H2)
    if H2 > 1:
        cols(x5[0:H2 - 1, 1], 0, 1, H2 - 1)

    wo = lax.broadcasted_iota(jnp.int32, (W_out, W_in), 0)
    wi = lax.broadcasted_iota(jnp.int32, (W_out, W_in), 1)
    sel = (wi == 2 * wo).astype(jnp.float32)
    selb = jnp.broadcast_to(sel[None], (H2, W_out, W_in))
    out = lax.dot_general(selb, acc_ref[...],
                          dimension_numbers=(((2,), (1,)), ((0,), (0,))),
                          preferred_element_type=jnp.float32)
    o_ref[0] = out.astype(o_ref.dtype)


def _conv3x3_s2(x, w9, b, apply_relu=False):
    N, H_in, W_in, Cin = x.shape
    Cout = w9.shape[-1]
    H2 = H_in // 2
    W_out = (W_in - 1) // 2 + 1
    body = functools.partial(_c3s2_kernel, H2=H2, W_in=W_in, W_out=W_out,
                             Cout=Cout, apply_relu=apply_relu)
    return pl.pallas_call(
        body,
        out_shape=jax.ShapeDtypeStruct((N, H2, W_out, Cout), jnp.float32),
        grid=(N,),
        in_specs=[
            pl.BlockSpec((1, H_in, W_in, Cin), lambda n: (n, 0, 0, 0)),
            pl.BlockSpec((9, Cin, Cout), lambda n: (0, 0, 0)),
            pl.BlockSpec((1, Cout), lambda n: (0, 0)),
        ],
        out_specs=pl.BlockSpec((1, H2, W_out, Cout), lambda n: (n, 0, 0, 0)),
        scratch_shapes=[pltpu.VMEM((H2, W_in, Cout), jnp.float32)],
        compiler_params=pltpu.CompilerParams(
            dimension_semantics=("parallel",),
            vmem_limit_bytes=_VMEM_LIMIT),
    )(x, w9, b.reshape(1, Cout))


def kernel(C3, C4, C5, P5_1_w, P5_1_b, P5_2_w, P5_2_b, P4_1_w, P4_1_b,
           P4_2_w, P4_2_b, P3_1_w, P3_1_b, P3_2_w, P3_2_b, P6_w, P6_b,
           P7_2_w, P7_2_b):
    to_nhwc = lambda t: jnp.transpose(t.astype(_BF), (0, 2, 3, 1))
    c3 = to_nhwc(C3)
    c4 = to_nhwc(C4)
    c5 = to_nhwc(C5)

    p5_lat = _conv1x1(c5, P5_1_w.astype(_BF), P5_1_b)
    p5 = _conv3x3_s1(p5_lat.astype(_BF), P5_2_w.astype(_BF), P5_2_b, 10)

    p4_lat = _conv1x1_upsample_add(c4, P4_1_w.astype(_BF), P4_1_b,
                                   p5_lat.astype(_BF))
    p4 = _conv3x3_s1(p4_lat.astype(_BF), P4_2_w.astype(_BF), P4_2_b, 10)

    p3_lat = _conv1x1_upsample_add(c3, P3_1_w.astype(_BF), P3_1_b,
                                   p4_lat.astype(_BF))
    p3 = _conv3x3_s1(p3_lat.astype(_BF), P3_2_w.astype(_BF), P3_2_b, 10)

    p6 = _conv3x3_s2(c5, P6_w.astype(_BF), P6_b)
    p7 = _conv3x3_s2(p6.astype(_BF), P7_2_w.astype(_BF), P7_2_b,
                     apply_relu=True)

    to_nchw = lambda t: jnp.transpose(t, (0, 3, 1, 2))
    return [to_nchw(p3), to_nchw(p4), to_nchw(p5), to_nchw(p6), to_nchw(p7)]
